# baseline (speedup 1.0000x reference)
import functools
import jax, jax.numpy as jnp
from jax import lax
from jax.experimental import pallas as pl
from jax.experimental.pallas import tpu as pltpu
from jax.experimental.pallas import tpu_sc as plsc

@functools.partial(
    pl.kernel,
    mesh=plsc.VectorSubcoreMesh(core_axis_name="c", subcore_axis_name="s"),
    compiler_params=pltpu.CompilerParams(use_tc_tiling_on_sc=True),
    out_type=jax.ShapeDtypeStruct((4096, 200, 64), jnp.float32),
    scratch_types=[
        pltpu.VMEM((200, 128), jnp.int32),
        pltpu.VMEM((128, 64), jnp.float32),
        pltpu.VMEM((128, 64), jnp.float32),
        pltpu.SemaphoreType.DMA,
        pltpu.SemaphoreType.DMA,
        pltpu.SemaphoreType.DMA,
        pltpu.SemaphoreType.DMA,
    ],
)
def _k(xt_hbm, table_hbm, out_hbm, idx_v, rows0, rows1, g0, g1, s0, s1):
    rows = (rows0, rows1)
    gsem = (g0, g1)
    ssem = (s0, s1)
    wid = lax.axis_index("s") * 2 + lax.axis_index("c")
    b0 = wid * 128
    pltpu.sync_copy(xt_hbm.at[:, pl.ds(b0, 128)], idx_v)

    def start_gathers(t, slot):
        def bg_body(bg, c2):
            vec = idx_v[t, pl.ds(bg * 16, 16)]
            for k in range(16):
                vk = vec[k]
                pltpu.async_copy(
                    table_hbm.at[vk], rows[slot].at[bg * 16 + k], gsem[slot]
                )
            return c2
        lax.fori_loop(0, 8, bg_body, 0)

    def wait_gathers(t, slot):
        def w_body(j, c2):
            pltpu.make_async_copy(
                table_hbm.at[0], rows[slot].at[0], gsem[slot]
            ).wait()
            return c2
        lax.fori_loop(0, 128, w_body, 0)

    def start_store(t, slot):
        pltpu.async_copy(rows[slot], out_hbm.at[pl.ds(b0, 128), t], ssem[slot])

    def wait_store(t, slot):
        pltpu.make_async_copy(
            rows[slot], out_hbm.at[pl.ds(b0, 128), t], ssem[slot]
        ).wait()

    start_gathers(0, 0)
    start_gathers(1, 1)

    def pair_body(p, carry):
        for slot in (0, 1):
            t = p * 2 + slot
            wait_gathers(t, slot)

            start_store(t, slot)

            @pl.when(t + 2 < 200)
            def _(slot=slot, t=t):
                wait_store(t, slot)
                start_gathers(t + 2, slot)
        return carry

    lax.fori_loop(0, 100, pair_body, 0)
    wait_store(198, 0)
    wait_store(199, 1)

def kernel(x, table):
    xt = x.T.astype(jnp.int32)
    return _k(xt, table)


# speedup vs baseline: 1.0616x; 1.0616x over previous
import functools
import jax, jax.numpy as jnp
from jax import lax
from jax.experimental import pallas as pl
from jax.experimental.pallas import tpu as pltpu
from jax.experimental.pallas import tpu_sc as plsc

@functools.partial(
    pl.kernel,
    mesh=plsc.VectorSubcoreMesh(core_axis_name="c", subcore_axis_name="s"),
    compiler_params=pltpu.CompilerParams(use_tc_tiling_on_sc=True),
    out_type=jax.ShapeDtypeStruct((4096, 200, 64), jnp.float32),
    scratch_types=[
        pltpu.VMEM((200, 128), jnp.int32),
        pltpu.VMEM((128, 64), jnp.float32),
        pltpu.VMEM((128, 64), jnp.float32),
        pltpu.SemaphoreType.DMA,
        pltpu.SemaphoreType.DMA,
        pltpu.SemaphoreType.DMA,
        pltpu.SemaphoreType.DMA,
    ],
)
def _k(xt_hbm, table_hbm, out_hbm, idx_v, rows0, rows1, g0, g1, s0, s1):
    rows = (rows0, rows1)
    gsem = (g0, g1)
    ssem = (s0, s1)
    wid = lax.axis_index("s") * 2 + lax.axis_index("c")
    b0 = wid * 128
    pltpu.sync_copy(xt_hbm.at[:, pl.ds(b0, 128)], idx_v)

    def start_gathers(t, slot):
        for bg in range(8):
            vec = idx_v[t, pl.ds(bg * 16, 16)]
            for k in range(16):
                vk = vec[k]
                pltpu.async_copy(
                    table_hbm.at[vk], rows[slot].at[bg * 16 + k], gsem[slot]
                )

    def wait_gathers(t, slot):
        pltpu.make_async_copy(
            table_hbm.at[pl.ds(0, 128), :], rows[slot], gsem[slot]
        ).wait()

    def start_store(t, slot):
        pltpu.async_copy(rows[slot], out_hbm.at[pl.ds(b0, 128), t], ssem[slot])

    def wait_store(t, slot):
        pltpu.make_async_copy(
            rows[slot], out_hbm.at[pl.ds(b0, 128), t], ssem[slot]
        ).wait()

    start_gathers(0, 0)
    start_gathers(1, 1)

    def pair_body(p, carry):
        for slot in (0, 1):
            t = p * 2 + slot
            wait_gathers(t, slot)

            start_store(t, slot)

            @pl.when(t + 2 < 200)
            def _(slot=slot, t=t):
                wait_store(t, slot)
                start_gathers(t + 2, slot)
        return carry

    lax.fori_loop(0, 100, pair_body, 0)
    wait_store(198, 0)
    wait_store(199, 1)

def kernel(x, table):
    xt = x.T.astype(jnp.int32)
    return _k(xt, table)
